# baseline (speedup 1.0000x reference)
import jax
import jax.numpy as jnp
from jax.experimental import pallas as pl
from jax.experimental.pallas import tpu as pltpu

EMBED = 64
NUM_CODES = 1024
ROWS = 16 * 1024
BLK = 1024
GRID = ROWS // BLK
_LOSS_SCALE = 1.25 / (ROWS * EMBED)


def _vq_block(z_ref, e_ref, zq_ref, loss_ref, acc_ref):
    i = pl.program_id(0)
    z = z_ref[...]
    e = e_ref[...]
    z2 = jnp.sum(z * z, axis=1, keepdims=True)
    e2 = jnp.sum(e * e, axis=1)
    m = jax.lax.dot_general(
        z, e, (((1,), (1,)), ((), ())),
        preferred_element_type=jnp.float32)
    d = (z2 + e2[None, :]) - 2.0 * m
    mval = jnp.min(d, axis=1, keepdims=True)
    kiota = jax.lax.broadcasted_iota(jnp.int32, d.shape, 1)
    idx = jnp.min(jnp.where(d == mval, kiota, NUM_CODES),
                  axis=1, keepdims=True)
    onehot = (kiota == idx).astype(jnp.float32)
    zq = jax.lax.dot_general(
        onehot, e, (((1,), (0,)), ((), ())),
        preferred_element_type=jnp.float32)
    zq_ref[...] = z + (zq - z)

    part = jnp.sum((z - zq) ** 2)

    @pl.when(i == 0)
    def _init():
        acc_ref[0] = 0.0

    acc_ref[0] += part

    @pl.when(i == GRID - 1)
    def _fin():
        loss_ref[0, 0] = acc_ref[0] * _LOSS_SCALE


def kernel(z_e, embeddings):
    zf = z_e.reshape(ROWS, EMBED)
    zq_st, loss = pl.pallas_call(
        _vq_block,
        grid=(GRID,),
        in_specs=[
            pl.BlockSpec((BLK, EMBED), lambda i: (i, 0)),
            pl.BlockSpec((NUM_CODES, EMBED), lambda i: (0, 0)),
        ],
        out_specs=[
            pl.BlockSpec((BLK, EMBED), lambda i: (i, 0)),
            pl.BlockSpec((1, 1), lambda i: (0, 0), memory_space=pltpu.SMEM),
        ],
        out_shape=[
            jax.ShapeDtypeStruct((ROWS, EMBED), jnp.float32),
            jax.ShapeDtypeStruct((1, 1), jnp.float32),
        ],
        scratch_shapes=[pltpu.SMEM((1,), jnp.float32)],
    )(zf, embeddings)
    return zq_st.reshape(z_e.shape), loss.reshape(())


# speedup vs baseline: 1.1327x; 1.1327x over previous
import jax
import jax.numpy as jnp
from jax.experimental import pallas as pl
from jax.experimental.pallas import tpu as pltpu

EMBED = 64
NUM_CODES = 1024
ROWS = 16 * 1024
BLK = 2048
GRID = ROWS // BLK
_LOSS_SCALE = 1.25 / (ROWS * EMBED)


def _vq_block(z_ref, e_ref, zq_ref, loss_ref, e2_ref, kio_ref, acc_ref):
    i = pl.program_id(0)
    z = z_ref[...]
    e = e_ref[...]

    @pl.when(i == 0)
    def _init():
        e2_ref[0, :] = jnp.sum(e * e, axis=1)
        kio_ref[...] = jax.lax.broadcasted_iota(
            jnp.int32, (1, NUM_CODES), 1).astype(jnp.float32)
        acc_ref[0] = 0.0

    z2 = jnp.sum(z * z, axis=1, keepdims=True)
    m = jax.lax.dot_general(
        z, e, (((1,), (1,)), ((), ())),
        preferred_element_type=jnp.float32)
    d = (z2 + e2_ref[...]) - 2.0 * m
    mval = jnp.min(d, axis=1, keepdims=True)
    kiota = jnp.broadcast_to(kio_ref[...], d.shape)
    u = jnp.where(d == mval, kiota, jnp.float32(NUM_CODES))
    umin = jnp.min(u, axis=1, keepdims=True)
    onehot = jnp.where(kiota == umin, 1.0, 0.0)
    zq = jax.lax.dot_general(
        onehot, e, (((1,), (0,)), ((), ())),
        preferred_element_type=jnp.float32)
    zq_ref[...] = z + (zq - z)
    acc_ref[0] += jnp.sum(mval)

    @pl.when(i == GRID - 1)
    def _fin():
        loss_ref[0, 0] = acc_ref[0] * _LOSS_SCALE


def kernel(z_e, embeddings):
    zf = z_e.reshape(ROWS, EMBED)
    zq_st, loss = pl.pallas_call(
        _vq_block,
        grid=(GRID,),
        in_specs=[
            pl.BlockSpec((BLK, EMBED), lambda i: (i, 0)),
            pl.BlockSpec((NUM_CODES, EMBED), lambda i: (0, 0)),
        ],
        out_specs=[
            pl.BlockSpec((BLK, EMBED), lambda i: (i, 0)),
            pl.BlockSpec((1, 1), lambda i: (0, 0), memory_space=pltpu.SMEM),
        ],
        out_shape=[
            jax.ShapeDtypeStruct((ROWS, EMBED), jnp.float32),
            jax.ShapeDtypeStruct((1, 1), jnp.float32),
        ],
        scratch_shapes=[
            pltpu.VMEM((1, NUM_CODES), jnp.float32),
            pltpu.VMEM((1, NUM_CODES), jnp.float32),
            pltpu.SMEM((1,), jnp.float32),
        ],
    )(zf, embeddings)
    return zq_st.reshape(z_e.shape), loss.reshape(())
